# baseline (speedup 1.0000x reference)
import functools
import math

import jax
import jax.numpy as jnp
from jax.experimental import pallas as pl
from jax.experimental.pallas import tpu as pltpu

N_HEADS = 16
N_KV_HEADS = 4
HEAD_SIZE = 128
MAX_SEQLEN = 2048
SPARSE_BLOCK = 64
VLLM_BLOCK = 16
LOCAL_BLOCKS = 8
VERT_STRIDE = 4
NUM_SEQS = 32
BLOCKS_PER_SEQ = MAX_SEQLEN // VLLM_BLOCK
NUM_SPARSE_BLOCKS = MAX_SEQLEN // SPARSE_BLOCK
VPB = SPARSE_BLOCK // VLLM_BLOCK
MAX_SLOTS = 14
SM_SCALE = 1.0 / math.sqrt(HEAD_SIZE)
NEG_INF = -1e30


def _attn_body(cb_ref, lim_ref, q_ref,
               k0, k1, k2, k3, v0, v1, v2, v3,
               o_ref, m_scr, l_scr, acc_scr):
    s = pl.program_id(0)
    slot = pl.program_id(1)

    @pl.when(slot == 0)
    def _init():
        m_scr[...] = jnp.full((N_HEADS, HEAD_SIZE), NEG_INF, jnp.float32)
        l_scr[...] = jnp.zeros((N_HEADS, HEAD_SIZE), jnp.float32)
        acc_scr[...] = jnp.zeros((N_HEADS, HEAD_SIZE), jnp.float32)

    k_cat = jnp.concatenate([k0[0], k1[0], k2[0], k3[0]], axis=-1)
    v_cat = jnp.concatenate([v0[0], v1[0], v2[0], v3[0]], axis=-1)

    q4 = q_ref[0].reshape(N_KV_HEADS, N_HEADS // N_KV_HEADS, HEAD_SIZE)
    sc = jax.lax.dot_general(
        q4, k_cat,
        dimension_numbers=(((2,), (1,)), ((0,), (0,))),
        preferred_element_type=jnp.float32,
    ).reshape(N_HEADS, SPARSE_BLOCK) * SM_SCALE

    limit = lim_ref[s, slot]
    tok = jax.lax.broadcasted_iota(jnp.int32, (N_HEADS, SPARSE_BLOCK), 1)
    sc = jnp.where(tok < limit, sc, NEG_INF)

    m_old = m_scr[...]
    m_new = jnp.maximum(m_old, jnp.max(sc, axis=-1, keepdims=True))
    alpha = jnp.exp(m_old - m_new)
    p = jnp.exp(sc - m_new[:, :SPARSE_BLOCK])
    l_scr[...] = l_scr[...] * alpha + jnp.sum(p, axis=-1, keepdims=True)
    pv = jax.lax.dot_general(
        p.reshape(N_KV_HEADS, N_HEADS // N_KV_HEADS, SPARSE_BLOCK), v_cat,
        dimension_numbers=(((2,), (2,)), ((0,), (0,))),
        preferred_element_type=jnp.float32,
    ).reshape(N_HEADS, HEAD_SIZE)
    acc_scr[...] = acc_scr[...] * alpha + pv
    m_scr[...] = m_new

    @pl.when(slot == MAX_SLOTS - 1)
    def _finish():
        o_ref[0] = acc_scr[...] / l_scr[...]


def _routing(block_tables, context_lens):
    qblk = (context_lens - 1) // SPARSE_BLOCK
    j = jnp.arange(NUM_SPARSE_BLOCKS, dtype=jnp.int32)
    vis = (j[None, :] <= qblk[:, None]) & (
        (qblk[:, None] - j[None, :] < LOCAL_BLOCKS)
        | ((j[None, :] + 1) % VERT_STRIDE == 0))
    key = jnp.where(vis, j[None, :], jnp.int32(10_000))
    packed = jnp.sort(key, axis=1)[:, :MAX_SLOTS]
    counts = jnp.sum(vis.astype(jnp.int32), axis=1)
    slot = jnp.arange(MAX_SLOTS, dtype=jnp.int32)
    valid = slot[None, :] < counts[:, None]
    visj = jnp.where(valid, packed, qblk[:, None])
    lim = jnp.where(
        valid,
        jnp.clip(context_lens[:, None] - SPARSE_BLOCK * visj, 0, SPARSE_BLOCK),
        0).astype(jnp.int32)
    vb = (VPB * visj[:, :, None]
          + jnp.arange(VPB, dtype=jnp.int32)[None, None, :]).reshape(
              NUM_SEQS, MAX_SLOTS * VPB)
    cb = jnp.take_along_axis(block_tables, vb, axis=1)
    return cb.astype(jnp.int32), lim


@jax.jit
def kernel(q, k, v, block_tables, context_lens):
    cb, lim = _routing(block_tables, context_lens)

    def q_map(s, t, cb_ref, lim_ref):
        return (s, 0, 0)

    def kv_map(i, s, t, cb_ref, lim_ref):
        return (cb_ref[s, VPB * t + i], 0, 0, 0)

    kv_spec = lambda i: pl.BlockSpec(
        (1, N_KV_HEADS, HEAD_SIZE, VLLM_BLOCK), functools.partial(kv_map, i))

    grid_spec = pltpu.PrefetchScalarGridSpec(
        num_scalar_prefetch=2,
        grid=(NUM_SEQS, MAX_SLOTS),
        in_specs=[
            pl.BlockSpec((1, N_HEADS, HEAD_SIZE), q_map),
            kv_spec(0), kv_spec(1), kv_spec(2), kv_spec(3),
            kv_spec(0), kv_spec(1), kv_spec(2), kv_spec(3),
        ],
        out_specs=pl.BlockSpec(
            (1, N_HEADS, HEAD_SIZE), lambda s, t, cb_ref, lim_ref: (s, 0, 0)),
        scratch_shapes=[
            pltpu.VMEM((N_HEADS, HEAD_SIZE), jnp.float32),
            pltpu.VMEM((N_HEADS, HEAD_SIZE), jnp.float32),
            pltpu.VMEM((N_HEADS, HEAD_SIZE), jnp.float32),
        ],
    )

    out = pl.pallas_call(
        _attn_body,
        grid_spec=grid_spec,
        out_shape=jax.ShapeDtypeStruct((NUM_SEQS, N_HEADS, HEAD_SIZE),
                                       jnp.float32),
        compiler_params=pltpu.CompilerParams(
            dimension_semantics=("arbitrary", "arbitrary")),
    )(cb, lim, q, k, k, k, k, v, v, v, v)
    return out


# speedup vs baseline: 1.0817x; 1.0817x over previous
import functools
import math

import jax
import jax.numpy as jnp
from jax.experimental import pallas as pl
from jax.experimental.pallas import tpu as pltpu

N_HEADS = 16
N_KV_HEADS = 4
HEAD_SIZE = 128
MAX_SEQLEN = 2048
SPARSE_BLOCK = 64
VLLM_BLOCK = 16
LOCAL_BLOCKS = 8
VERT_STRIDE = 4
NUM_SEQS = 32
BLOCKS_PER_SEQ = MAX_SEQLEN // VLLM_BLOCK
NUM_SPARSE_BLOCKS = MAX_SEQLEN // SPARSE_BLOCK
VPB = SPARSE_BLOCK // VLLM_BLOCK
MAX_SLOTS = 14
NUM_VB = MAX_SLOTS * VPB
SM_SCALE = 1.0 / math.sqrt(HEAD_SIZE)
NEG_INF = -1e30


def _attn_body(cb_ref, lim_ref, q_ref, *refs):
    k_refs = refs[:NUM_VB]
    v_refs = refs[NUM_VB:2 * NUM_VB]
    o_ref = refs[2 * NUM_VB]
    s = pl.program_id(0)

    q4 = q_ref[0].reshape(N_KV_HEADS, N_HEADS // N_KV_HEADS, HEAD_SIZE)
    tok = jax.lax.broadcasted_iota(jnp.int32, (N_HEADS, SPARSE_BLOCK), 1)

    chunks = []
    for i in range(MAX_SLOTS):
        k_cat = jnp.concatenate(
            [k_refs[VPB * i + u][0] for u in range(VPB)], axis=-1)
        sc_i = jax.lax.dot_general(
            q4, k_cat,
            dimension_numbers=(((2,), (1,)), ((0,), (0,))),
            preferred_element_type=jnp.float32,
        ).reshape(N_HEADS, SPARSE_BLOCK) * SM_SCALE
        bias_i = jnp.where(tok < lim_ref[s, i], 0.0, NEG_INF).astype(
            jnp.float32)
        chunks.append(sc_i + bias_i)
    sc = jnp.concatenate(chunks, axis=-1)

    m = jnp.max(sc, axis=-1, keepdims=True)
    p = jnp.exp(sc - m)
    l = jnp.sum(p, axis=-1, keepdims=True)

    pv = jnp.zeros((N_KV_HEADS, N_HEADS // N_KV_HEADS, HEAD_SIZE), jnp.float32)
    for i in range(MAX_SLOTS):
        v_cat = jnp.concatenate(
            [v_refs[VPB * i + u][0] for u in range(VPB)], axis=-1)
        p_i = p[:, i * SPARSE_BLOCK:(i + 1) * SPARSE_BLOCK].reshape(
            N_KV_HEADS, N_HEADS // N_KV_HEADS, SPARSE_BLOCK)
        pv = pv + jax.lax.dot_general(
            p_i, v_cat,
            dimension_numbers=(((2,), (2,)), ((0,), (0,))),
            preferred_element_type=jnp.float32,
        )
    o_ref[0] = pv.reshape(N_HEADS, HEAD_SIZE) / l


def _routing(block_tables, context_lens):
    qblk = (context_lens - 1) // SPARSE_BLOCK
    j = jnp.arange(NUM_SPARSE_BLOCKS, dtype=jnp.int32)
    vis = (j[None, :] <= qblk[:, None]) & (
        (qblk[:, None] - j[None, :] < LOCAL_BLOCKS)
        | ((j[None, :] + 1) % VERT_STRIDE == 0))
    key = jnp.where(vis, j[None, :], jnp.int32(10_000))
    packed = jnp.sort(key, axis=1)[:, :MAX_SLOTS]
    counts = jnp.sum(vis.astype(jnp.int32), axis=1)
    slot = jnp.arange(MAX_SLOTS, dtype=jnp.int32)
    valid = slot[None, :] < counts[:, None]
    visj = jnp.where(valid, packed, qblk[:, None])
    lim = jnp.where(
        valid,
        jnp.clip(context_lens[:, None] - SPARSE_BLOCK * visj, 0, SPARSE_BLOCK),
        0).astype(jnp.int32)
    vb = (VPB * visj[:, :, None]
          + jnp.arange(VPB, dtype=jnp.int32)[None, None, :]).reshape(
              NUM_SEQS, NUM_VB)
    cb = jnp.take_along_axis(block_tables, vb, axis=1)
    return cb.astype(jnp.int32), lim


@jax.jit
def kernel(q, k, v, block_tables, context_lens):
    cb, lim = _routing(block_tables, context_lens)

    def kv_map(i, s, cb_ref, lim_ref):
        return (cb_ref[s, i], 0, 0, 0)

    kv_spec = lambda i: pl.BlockSpec(
        (1, N_KV_HEADS, HEAD_SIZE, VLLM_BLOCK), functools.partial(kv_map, i))

    grid_spec = pltpu.PrefetchScalarGridSpec(
        num_scalar_prefetch=2,
        grid=(NUM_SEQS,),
        in_specs=[
            pl.BlockSpec((1, N_HEADS, HEAD_SIZE),
                         lambda s, cb_ref, lim_ref: (s, 0, 0)),
        ] + [kv_spec(i) for i in range(NUM_VB)] * 2,
        out_specs=pl.BlockSpec(
            (1, N_HEADS, HEAD_SIZE), lambda s, cb_ref, lim_ref: (s, 0, 0)),
        scratch_shapes=[],
    )

    out = pl.pallas_call(
        _attn_body,
        grid_spec=grid_spec,
        out_shape=jax.ShapeDtypeStruct((NUM_SEQS, N_HEADS, HEAD_SIZE),
                                       jnp.float32),
        compiler_params=pltpu.CompilerParams(
            dimension_semantics=("arbitrary",)),
    )(cb, lim, q, *([k] * NUM_VB), *([v] * NUM_VB))
    return out
